# Initial kernel scaffold; baseline (speedup 1.0000x reference)
import functools

import jax
import jax.numpy as jnp
from jax import lax
from jax.experimental import pallas as pl
from jax.experimental.pallas import tpu as pltpu
from jax.experimental.pallas import tpu_sc as plsc

_SC_CORES = 2
_SC_SUBCORES = 16
_NW = _SC_CORES * _SC_SUBCORES

_GATHER_CHUNK = 128


def _sc_gather_rows(table, idx):
    R, D = table.shape
    N = idx.shape[0]
    per_w = N // _NW
    n_chunks = per_w // _GATHER_CHUNK
    assert per_w % _GATHER_CHUNK == 0 and N % (8 * _NW) == 0

    mesh = plsc.VectorSubcoreMesh(core_axis_name="c", subcore_axis_name="s")

    @functools.partial(
        pl.kernel,
        mesh=mesh,
        out_type=jax.ShapeDtypeStruct((N, D), jnp.float32),
        scratch_types=[
            pltpu.VMEM((_GATHER_CHUNK,), jnp.int32),
            pltpu.VMEM((_GATHER_CHUNK, D), jnp.float32),
            pltpu.SemaphoreType.DMA,
        ],
    )
    def gather_kernel(table_hbm, idx_hbm, out_hbm, idx_v, rows_v, sem):
        wid = lax.axis_index("s") * _SC_CORES + lax.axis_index("c")
        base = wid * per_w
        for j in range(n_chunks):
            off = base + j * _GATHER_CHUNK
            pltpu.sync_copy(idx_hbm.at[pl.ds(off, _GATHER_CHUNK)], idx_v)
            pltpu.async_copy(table_hbm.at[idx_v], rows_v, sem).wait()
            pltpu.sync_copy(rows_v, out_hbm.at[pl.ds(off, _GATHER_CHUNK)])

    return gather_kernel(table, idx)


_BT = 256


def _mlp_kernel(h_ref, colf_ref, motif_ref, ct_ref, w1h_ref, w1c_ref,
                wm_ref, b1_ref, w2_ref, b2_ref, out_ref):
    z = lax.dot_general(h_ref[...], w1h_ref[...], (((1,), (1,)), ((), ())),
                        preferred_element_type=jnp.float32)
    z += lax.dot_general(colf_ref[...], w1c_ref[...], (((1,), (1,)), ((), ())),
                         preferred_element_type=jnp.float32)
    z += motif_ref[...] * wm_ref[...] + b1_ref[...]
    hm = jnp.maximum(z, 0.0)
    logit = lax.dot_general(hm, w2_ref[...], (((1,), (1,)), ((), ())),
                            preferred_element_type=jnp.float32)
    logit = logit + b2_ref[0, 0]
    valid = ct_ref[...] >= 0
    out_ref[...] = jnp.where(valid, logit.reshape(1, _BT), 0.0)


def kernel(hidden_states, column_features, W1, b1, W2, b2, c_t, motif_indicators):
    B, T, H = hidden_states.shape
    _, NC, FD = column_features.shape
    N = B * T

    c_safe = jnp.where(c_t >= 0, c_t, 0)
    flat_idx = (jnp.arange(B, dtype=jnp.int32)[:, None] * NC + c_safe).reshape(N)

    colf = _sc_gather_rows(column_features.reshape(B * NC, FD), flat_idx)

    h2 = hidden_states.reshape(N, H)
    motif = motif_indicators.reshape(N, 1).astype(jnp.float32)
    ct2 = c_t.reshape(N // _BT, _BT)

    W1h = W1[:, :H]
    W1c = W1[:, H:H + FD]
    wm = W1[:, H + FD].reshape(1, H)
    b1r = b1.reshape(1, H)
    b2r = b2.reshape(1, 1)

    grid = (N // _BT,)
    out = pl.pallas_call(
        _mlp_kernel,
        grid=grid,
        in_specs=[
            pl.BlockSpec((_BT, H), lambda i: (i, 0)),
            pl.BlockSpec((_BT, FD), lambda i: (i, 0)),
            pl.BlockSpec((_BT, 1), lambda i: (i, 0)),
            pl.BlockSpec((1, _BT), lambda i: (i, 0)),
            pl.BlockSpec((H, H), lambda i: (0, 0)),
            pl.BlockSpec((H, FD), lambda i: (0, 0)),
            pl.BlockSpec((1, H), lambda i: (0, 0)),
            pl.BlockSpec((1, H), lambda i: (0, 0)),
            pl.BlockSpec((1, H), lambda i: (0, 0)),
            pl.BlockSpec((1, 1), lambda i: (0, 0)),
        ],
        out_specs=pl.BlockSpec((1, _BT), lambda i: (i, 0)),
        out_shape=jax.ShapeDtypeStruct((N // _BT, _BT), jnp.float32),
    )(h2, colf, motif, ct2, W1h, W1c, wm, b1r, W2, b2r)

    return out.reshape(B, T)


# speedup vs baseline: 1.6984x; 1.6984x over previous
import functools

import jax
import jax.numpy as jnp
from jax import lax
from jax.experimental import pallas as pl
from jax.experimental.pallas import tpu as pltpu
from jax.experimental.pallas import tpu_sc as plsc

_SC_CORES = 2
_SC_SUBCORES = 16
_NW = _SC_CORES * _SC_SUBCORES

_GATHER_CHUNK = 128


def _sc_gather_rows(table, idx):
    R, D = table.shape
    N = idx.shape[0]
    per_w = N // _NW
    n_chunks = per_w // _GATHER_CHUNK
    assert per_w % _GATHER_CHUNK == 0 and N % (8 * _NW) == 0

    mesh = plsc.VectorSubcoreMesh(core_axis_name="c", subcore_axis_name="s")

    @functools.partial(
        pl.kernel,
        mesh=mesh,
        out_type=jax.ShapeDtypeStruct((N, D), jnp.float32),
        scratch_types=[
            pltpu.VMEM((_GATHER_CHUNK,), jnp.int32),
            pltpu.VMEM((_GATHER_CHUNK, D), jnp.float32),
            pltpu.SemaphoreType.DMA,
        ],
    )
    def gather_kernel(table_hbm, idx_hbm, out_hbm, idx_v, rows_v, sem):
        wid = lax.axis_index("s") * _SC_CORES + lax.axis_index("c")
        base = wid * per_w
        for j in range(n_chunks):
            off = base + j * _GATHER_CHUNK
            pltpu.sync_copy(idx_hbm.at[pl.ds(off, _GATHER_CHUNK)], idx_v)
            pltpu.async_copy(table_hbm.at[idx_v], rows_v, sem).wait()
            pltpu.sync_copy(rows_v, out_hbm.at[pl.ds(off, _GATHER_CHUNK)])

    return gather_kernel(table, idx)


_BT = 256


def _mlp_kernel(h_ref, colf_ref, motif_ref, ct_ref, w1h_ref, w1c_ref,
                wm_ref, b1_ref, w2_ref, b2_ref, out_ref):
    z = lax.dot_general(h_ref[...], w1h_ref[...], (((1,), (1,)), ((), ())),
                        preferred_element_type=jnp.float32)
    z += lax.dot_general(colf_ref[...], w1c_ref[...], (((1,), (1,)), ((), ())),
                         preferred_element_type=jnp.float32)
    z += motif_ref[...] * wm_ref[...] + b1_ref[...]
    hm = jnp.maximum(z, 0.0)
    logit = jnp.sum(hm * w2_ref[...], axis=1, keepdims=True)
    logit = logit + b2_ref[0, 0]
    valid = ct_ref[...] >= 0
    out_ref[...] = jnp.where(valid, logit, 0.0)


def kernel(hidden_states, column_features, W1, b1, W2, b2, c_t, motif_indicators):
    B, T, H = hidden_states.shape
    _, NC, FD = column_features.shape
    N = B * T

    c_safe = jnp.where(c_t >= 0, c_t, 0)
    flat_idx = (jnp.arange(B, dtype=jnp.int32)[:, None] * NC + c_safe).reshape(N)

    colf = _sc_gather_rows(column_features.reshape(B * NC, FD), flat_idx)

    h2 = hidden_states.reshape(N, H)
    motif = motif_indicators.reshape(N, 1).astype(jnp.float32)
    ct2 = c_t.reshape(N, 1)

    W1h = W1[:, :H]
    W1c = W1[:, H:H + FD]
    wm = W1[:, H + FD].reshape(1, H)
    b1r = b1.reshape(1, H)
    b2r = b2.reshape(1, 1)

    grid = (N // _BT,)
    out = pl.pallas_call(
        _mlp_kernel,
        grid=grid,
        in_specs=[
            pl.BlockSpec((_BT, H), lambda i: (i, 0)),
            pl.BlockSpec((_BT, FD), lambda i: (i, 0)),
            pl.BlockSpec((_BT, 1), lambda i: (i, 0)),
            pl.BlockSpec((_BT, 1), lambda i: (i, 0)),
            pl.BlockSpec((H, H), lambda i: (0, 0)),
            pl.BlockSpec((H, FD), lambda i: (0, 0)),
            pl.BlockSpec((1, H), lambda i: (0, 0)),
            pl.BlockSpec((1, H), lambda i: (0, 0)),
            pl.BlockSpec((1, H), lambda i: (0, 0)),
            pl.BlockSpec((1, 1), lambda i: (0, 0)),
        ],
        out_specs=pl.BlockSpec((_BT, 1), lambda i: (i, 0)),
        out_shape=jax.ShapeDtypeStruct((N, 1), jnp.float32),
    )(h2, colf, motif, ct2, W1h, W1c, wm, b1r, W2, b2r)

    return out.reshape(B, T)


# speedup vs baseline: 1.7601x; 1.0363x over previous
import functools

import jax
import jax.numpy as jnp
from jax import lax
from jax.experimental import pallas as pl
from jax.experimental.pallas import tpu as pltpu
from jax.experimental.pallas import tpu_sc as plsc

_SC_CORES = 2
_SC_SUBCORES = 16
_NW = _SC_CORES * _SC_SUBCORES

_GATHER_CHUNK = 128


def _sc_gather_rows(table, idx):
    R, D = table.shape
    N = idx.shape[0]
    per_w = N // _NW
    n_chunks = per_w // _GATHER_CHUNK
    assert per_w % _GATHER_CHUNK == 0 and N % (8 * _NW) == 0

    mesh = plsc.VectorSubcoreMesh(core_axis_name="c", subcore_axis_name="s")

    @functools.partial(
        pl.kernel,
        mesh=mesh,
        out_type=jax.ShapeDtypeStruct((N, D), jnp.float32),
        scratch_types=[
            pltpu.VMEM((_GATHER_CHUNK,), jnp.int32),
            pltpu.VMEM((_GATHER_CHUNK, D), jnp.float32),
            pltpu.SemaphoreType.DMA,
        ],
    )
    def gather_kernel(table_hbm, idx_hbm, out_hbm, idx_v, rows_v, sem):
        wid = lax.axis_index("s") * _SC_CORES + lax.axis_index("c")
        base = wid * per_w
        for j in range(n_chunks):
            off = base + j * _GATHER_CHUNK
            pltpu.sync_copy(idx_hbm.at[pl.ds(off, _GATHER_CHUNK)], idx_v)
            pltpu.async_copy(table_hbm.at[idx_v], rows_v, sem).wait()
            pltpu.sync_copy(rows_v, out_hbm.at[pl.ds(off, _GATHER_CHUNK)])

    return gather_kernel(table, idx)


_BT = 256


def _mlp_kernel(h_ref, colf_ref, motif_ref, ct_ref, w1h_ref, w1c_ref,
                wm_ref, b1_ref, w2_ref, b2_ref, out_ref):
    z = lax.dot_general(h_ref[...].astype(jnp.bfloat16), w1h_ref[...],
                        (((1,), (1,)), ((), ())),
                        preferred_element_type=jnp.float32)
    z += lax.dot_general(colf_ref[...].astype(jnp.bfloat16), w1c_ref[...],
                         (((1,), (1,)), ((), ())),
                         preferred_element_type=jnp.float32)
    z += motif_ref[...] * wm_ref[...] + b1_ref[...]
    hm = jnp.maximum(z, 0.0)
    logit = jnp.sum(hm * w2_ref[...], axis=1, keepdims=True)
    logit = logit + b2_ref[0, 0]
    valid = ct_ref[...] >= 0
    out_ref[...] = jnp.where(valid, logit, 0.0)


def kernel(hidden_states, column_features, W1, b1, W2, b2, c_t, motif_indicators):
    B, T, H = hidden_states.shape
    _, NC, FD = column_features.shape
    N = B * T

    c_safe = jnp.where(c_t >= 0, c_t, 0)
    flat_idx = (jnp.arange(B, dtype=jnp.int32)[:, None] * NC + c_safe).reshape(N)

    colf = _sc_gather_rows(column_features.reshape(B * NC, FD), flat_idx)

    h2 = hidden_states.reshape(N, H)
    motif = motif_indicators.reshape(N, 1).astype(jnp.float32)
    ct2 = c_t.reshape(N, 1)

    W1h = W1[:, :H].astype(jnp.bfloat16)
    W1c = W1[:, H:H + FD].astype(jnp.bfloat16)
    wm = W1[:, H + FD].reshape(1, H)
    b1r = b1.reshape(1, H)
    b2r = b2.reshape(1, 1)

    grid = (N // _BT,)
    out = pl.pallas_call(
        _mlp_kernel,
        grid=grid,
        in_specs=[
            pl.BlockSpec((_BT, H), lambda i: (i, 0)),
            pl.BlockSpec((_BT, FD), lambda i: (i, 0)),
            pl.BlockSpec((_BT, 1), lambda i: (i, 0)),
            pl.BlockSpec((_BT, 1), lambda i: (i, 0)),
            pl.BlockSpec((H, H), lambda i: (0, 0)),
            pl.BlockSpec((H, FD), lambda i: (0, 0)),
            pl.BlockSpec((1, H), lambda i: (0, 0)),
            pl.BlockSpec((1, H), lambda i: (0, 0)),
            pl.BlockSpec((1, H), lambda i: (0, 0)),
            pl.BlockSpec((1, 1), lambda i: (0, 0)),
        ],
        out_specs=pl.BlockSpec((_BT, 1), lambda i: (i, 0)),
        out_shape=jax.ShapeDtypeStruct((N, 1), jnp.float32),
    )(h2, colf, motif, ct2, W1h, W1c, wm, b1r, W2, b2r)

    return out.reshape(B, T)
